# baseline (speedup 1.0000x reference)
import functools

import jax
import jax.numpy as jnp
from jax import lax
from jax.experimental import pallas as pl
from jax.experimental.pallas import tpu as pltpu
from jax.experimental.pallas import tpu_sc as plsc

N_NODES = 50000
K = 16
NLAT = 10
NBATCH = 4
MU = 1.0
NPAD = 53248
NB = 1024
NCHUNK = 128
SC_CHUNK = 128
SC_NCHUNKS = NPAD // SC_CHUNK
F32 = jnp.float32


def _encoder_body(w1_ref, xt_ref, b1_ref, w2_ref, b2_ref, wh0_ref, bh0_ref,
                  wh1_ref, bh1_ref, bp_ref, vt_ref, e_ref, hsc):
    i = pl.program_id(0)
    blk = jnp.dot(w1_ref[...], xt_ref[...], preferred_element_type=F32)
    hsc[pl.ds(i * 40, 40), :] = blk

    @pl.when(i == pl.num_programs(0) - 1)
    def _():
        h = hsc[...] + b1_ref[...]
        h = h * jax.nn.sigmoid(h)
        encT = jnp.dot(w2_ref[...], h, preferred_element_type=F32) + b2_ref[...]
        t0 = jnp.dot(wh0_ref[...], encT, preferred_element_type=F32) + bh0_ref[...]
        t0 = t0 * jax.nn.sigmoid(t0)
        t1 = jnp.dot(wh1_ref[...], t0, preferred_element_type=F32) + bh1_ref[...]
        hot = jax.nn.sigmoid(t1 * 0.005)
        logr = jnp.log(1.0 - 0.5 * hot)
        lane8 = lax.broadcasted_iota(jnp.int32, (8, 128), 1)
        sub8 = lax.broadcasted_iota(jnp.int32, (8, 128), 0)
        sel8 = ((lane8 // 32) == sub8).astype(F32)
        logr_l = jnp.dot(logr, sel8, preferred_element_type=F32)
        lane2 = lax.broadcasted_iota(jnp.int32, (64, 128), 1)
        e_exp = ((lane2 % 32) + 1).astype(F32)
        maskl = ((lane2 % 32) < NLAT).astype(F32)
        bp = bp_ref[0, 0]
        vt_ref[...] = jnp.exp(-2.0 * e_exp * logr_l) * (1.0 / (bp * MU) ** 2) * maskl
        li = lax.broadcasted_iota(jnp.int32, (128, NLAT), 0)
        ji = lax.broadcasted_iota(jnp.int32, (128, NLAT), 1)
        seljt = ((li % 32) == ji).astype(F32)
        e0 = jnp.dot(seljt, encT, preferred_element_type=F32)
        lb = lax.broadcasted_iota(jnp.int32, (128, 8), 0)
        bb = lax.broadcasted_iota(jnp.int32, (128, 8), 1)
        e_ref[...] = e0 * ((lb // 32) == bb).astype(F32)


def _encoder_call(w1, xt, b1c, w2, b2c, wh0, bh0c, wh1, bh1c, bpc):
    nsteps = 5
    const = lambda i: (0, 0)
    return pl.pallas_call(
        _encoder_body,
        grid=(nsteps,),
        in_specs=[
            pl.BlockSpec((40, N_NODES), lambda i: (i, 0)),
            pl.BlockSpec((N_NODES, 8), const),
            pl.BlockSpec((200, 1), const),
            pl.BlockSpec((NLAT, 200), const),
            pl.BlockSpec((NLAT, 1), const),
            pl.BlockSpec((128, NLAT), const),
            pl.BlockSpec((128, 1), const),
            pl.BlockSpec((64, 128), const),
            pl.BlockSpec((64, 1), const),
            pl.BlockSpec((1, 1), const),
        ],
        out_specs=[
            pl.BlockSpec((64, 128), const),
            pl.BlockSpec((128, 8), const),
        ],
        out_shape=[
            jax.ShapeDtypeStruct((64, 128), F32),
            jax.ShapeDtypeStruct((128, 8), F32),
        ],
        scratch_shapes=[pltpu.VMEM((200, 8), F32)],
    )(w1, xt, b1c, w2, b2c, wh0, bh0c, wh1, bh1c, bpc)


def _sc_body(wdec_hbm, nid2_hbm, dist_hbm, p_hbm, q_hbm,
             idx_v, dist_v, rows_v, p_v, q_v, sem):
    c = lax.axis_index("c")
    s = lax.axis_index("s")
    wid = s * 2 + c

    @pl.loop(0, SC_NCHUNKS // 32, unroll=1)
    def _chunk(t):
        ci = wid + 32 * t
        base = ci * SC_CHUNK
        row0 = ci * (SC_CHUNK * K // 128)
        pltpu.sync_copy(nid2_hbm.at[pl.ds(row0, SC_CHUNK * K // 128)], idx_v)
        pltpu.sync_copy(dist_hbm.at[pl.ds(base * K, SC_CHUNK * K)], dist_v)
        descs = [
            pltpu.async_copy(wdec_hbm.at[idx_v.at[g]],
                             rows_v.at[pl.ds(g * 128, 128)], sem)
            for g in range(SC_CHUNK * K // 128)
        ]
        for dsc in descs:
            dsc.wait()

        @pl.loop(0, SC_CHUNK, unroll=1)
        def _node(i):
            dvec = dist_v[pl.ds(i * K, K)]
            d2v = dvec * dvec
            p = jnp.zeros((16,), F32)
            q = jnp.zeros((16,), F32)
            for k in range(K):
                row = rows_v[i * K + k, :]
                p = p + row
                q = q + d2v[k] * row
            p_v[i, :] = p
            q_v[i, :] = q

        pltpu.sync_copy(p_v, p_hbm.at[pl.ds(base, SC_CHUNK)])
        pltpu.sync_copy(q_v, q_hbm.at[pl.ds(base, SC_CHUNK)])


def _sc_call(wdec_pad, nid2, distflat):
    mesh = plsc.VectorSubcoreMesh(core_axis_name="c", subcore_axis_name="s")
    assert SC_CHUNK * K // 128 % 8 == 0
    f = functools.partial(
        pl.kernel,
        out_type=[
            jax.ShapeDtypeStruct((NPAD, K), F32),
            jax.ShapeDtypeStruct((NPAD, K), F32),
        ],
        mesh=mesh,
        compiler_params=pltpu.CompilerParams(use_tc_tiling_on_sc=False),
        scratch_types=[
            pltpu.VMEM((SC_CHUNK * K // 128, 128), jnp.int32),
            pltpu.VMEM((SC_CHUNK * K,), F32),
            pltpu.VMEM((SC_CHUNK * K, K), F32),
            pltpu.VMEM((SC_CHUNK, K), F32),
            pltpu.VMEM((SC_CHUNK, K), F32),
            pltpu.SemaphoreType.DMA,
        ],
    )(_sc_body)
    return f(wdec_pad, nid2, distflat)


def _main_body(p_ref, q_ref, dist_ref, nid_ref, lab_ref, bdec_ref,
               vt_ref, e_ref, wdec_ref, out_ref):
    d = dist_ref[...]
    d2 = d * d
    ones16 = jnp.ones((K, 1), F32)
    s_sum = jnp.dot(d2, ones16, preferred_element_type=F32)
    lane64 = lax.broadcasted_iota(jnp.int32, (NB, 64), 1)
    oh = (lab_ref[...] == lane64).astype(F32)
    v = jnp.dot(oh, vt_ref[...], preferred_element_type=F32)
    sub16 = lax.broadcasted_iota(jnp.int32, (16, 128), 0)
    lane128 = lax.broadcasted_iota(jnp.int32, (16, 128), 1)
    repp = ((lane128 % 32) == sub16).astype(F32)
    clip = jnp.max(d2) * jnp.max(v) > 1.0

    @pl.when(jnp.logical_not(clip))
    def _fast():
        prep = jnp.dot(p_ref[...], repp, preferred_element_type=F32)
        qrep = jnp.dot(q_ref[...], repp, preferred_element_type=F32)
        conv = (prep - v * qrep) / (float(K) - v * s_sum)
        out_ref[...] = jnp.dot(conv, e_ref[...],
                               preferred_element_type=F32) + bdec_ref[...]

    @pl.when(clip)
    def _slow():
        accA = jnp.zeros((NB, 128), F32)
        accZ = jnp.zeros((NB, 128), F32)
        nid = nid_ref[...]
        lanech = lax.broadcasted_iota(jnp.int32, (NB, NCHUNK), 1)
        for k in range(K):
            idxk = nid[:, k:k + 1]

            def gk_body(cc, acc):
                wrow = wdec_ref[pl.ds(cc * NCHUNK, NCHUNK), :]
                ohc = (idxk == (cc * NCHUNK + lanech)).astype(F32)
                return acc + jnp.dot(ohc, wrow, preferred_element_type=F32)

            gk = lax.fori_loop(0, NPAD // NCHUNK, gk_body,
                               jnp.zeros((NB, K), F32))
            grep = jnp.dot(gk, repp, preferred_element_type=F32)
            win = jnp.maximum(1.0 - d2[:, k:k + 1] * v, 0.0)
            accZ = accZ + win
            accA = accA + win * grep
        conv = accA / accZ
        out_ref[...] = jnp.dot(conv, e_ref[...],
                               preferred_element_type=F32) + bdec_ref[...]


def _main_call(p_arr, q_arr, dist_pad, nid_pad, lab2, bdec2, vt, e_mat, wdec_pad):
    const = lambda i: (0, 0)
    blk = lambda i: (i, 0)
    return pl.pallas_call(
        _main_body,
        grid=(NPAD // NB,),
        in_specs=[
            pl.BlockSpec((NB, K), blk),
            pl.BlockSpec((NB, K), blk),
            pl.BlockSpec((NB, K), blk),
            pl.BlockSpec((NB, K), blk),
            pl.BlockSpec((NB, 1), blk),
            pl.BlockSpec((NB, 1), blk),
            pl.BlockSpec((64, 128), const),
            pl.BlockSpec((128, 8), const),
            pl.BlockSpec((NPAD, K), const),
        ],
        out_specs=pl.BlockSpec((NB, 8), blk),
        out_shape=jax.ShapeDtypeStruct((NPAD, 8), F32),
    )(p_arr, q_arr, dist_pad, nid_pad, lab2, bdec2, vt, e_mat, wdec_pad)


def kernel(x, W1, b1, W2, b2, Wdec, bdec, Wh0, bh0, Wh1, bh1, Bp,
           neighbour_distance, neighbour_id, clustering_labels):
    pad = NPAD - N_NODES
    xt = jnp.pad(x, ((0, 8 - NBATCH), (0, 0))).T
    wdec_pad = jnp.pad(Wdec, ((0, pad), (0, K - NLAT)))
    nid_pad = jnp.pad(neighbour_id, ((0, pad), (0, 0)))
    nid2 = nid_pad.reshape(NPAD * K // 128, 128)
    dist_pad = jnp.pad(neighbour_distance, ((0, pad), (0, 0)))
    distflat = dist_pad.reshape(NPAD * K)
    lab2 = jnp.pad(clustering_labels, (0, pad)).reshape(NPAD, 1)
    bdec2 = jnp.pad(bdec, (0, pad)).reshape(NPAD, 1)

    vt, e_mat = _encoder_call(
        W1, xt, b1.reshape(200, 1), W2, b2.reshape(NLAT, 1),
        Wh0, bh0.reshape(128, 1), Wh1, bh1.reshape(64, 1), Bp.reshape(1, 1))
    p_arr, q_arr = _sc_call(wdec_pad, nid2, distflat)
    out_t = _main_call(p_arr, q_arr, dist_pad, nid_pad, lab2, bdec2,
                       vt, e_mat, wdec_pad)
    return out_t[:N_NODES, :NBATCH].T


# speedup vs baseline: 1.5541x; 1.5541x over previous
import functools

import jax
import jax.numpy as jnp
from jax import lax
from jax.experimental import pallas as pl
from jax.experimental.pallas import tpu as pltpu
from jax.experimental.pallas import tpu_sc as plsc

N_NODES = 50000
K = 16
NLAT = 10
NBATCH = 4
MU = 1.0
NPAD = 50176
NB = 1024
NCHUNK = 128
SC_CHUNK = 128
SC_NCHUNKS = NPAD // SC_CHUNK
F32 = jnp.float32


def _encoder_body(w1_ref, xt_ref, b1_ref, w2_ref, b2_ref, wh0_ref, bh0_ref,
                  wh1_ref, bh1_ref, bp_ref, vt_ref, e_ref, hsc):
    i = pl.program_id(0)
    blk = jnp.dot(w1_ref[...], xt_ref[...], preferred_element_type=F32)
    hsc[pl.ds(i * 40, 40), :] = blk

    @pl.when(i == pl.num_programs(0) - 1)
    def _():
        h = hsc[...] + b1_ref[...]
        h = h * jax.nn.sigmoid(h)
        encT = jnp.dot(w2_ref[...], h, preferred_element_type=F32) + b2_ref[...]
        t0 = jnp.dot(wh0_ref[...], encT, preferred_element_type=F32) + bh0_ref[...]
        t0 = t0 * jax.nn.sigmoid(t0)
        t1 = jnp.dot(wh1_ref[...], t0, preferred_element_type=F32) + bh1_ref[...]
        hot = jax.nn.sigmoid(t1 * 0.005)
        logr = jnp.log(1.0 - 0.5 * hot)
        lane8 = lax.broadcasted_iota(jnp.int32, (8, 128), 1)
        sub8 = lax.broadcasted_iota(jnp.int32, (8, 128), 0)
        sel8 = ((lane8 // 32) == sub8).astype(F32)
        logr_l = jnp.dot(logr, sel8, preferred_element_type=F32)
        lane2 = lax.broadcasted_iota(jnp.int32, (64, 128), 1)
        e_exp = ((lane2 % 32) + 1).astype(F32)
        maskl = ((lane2 % 32) < NLAT).astype(F32)
        bp = bp_ref[0, 0]
        vt_ref[...] = jnp.exp(-2.0 * e_exp * logr_l) * (1.0 / (bp * MU) ** 2) * maskl
        li = lax.broadcasted_iota(jnp.int32, (128, NLAT), 0)
        ji = lax.broadcasted_iota(jnp.int32, (128, NLAT), 1)
        seljt = ((li % 32) == ji).astype(F32)
        e0 = jnp.dot(seljt, encT, preferred_element_type=F32)
        lb = lax.broadcasted_iota(jnp.int32, (128, 8), 0)
        bb = lax.broadcasted_iota(jnp.int32, (128, 8), 1)
        e_ref[...] = e0 * ((lb // 32) == bb).astype(F32)


def _encoder_call(w1, xt, b1c, w2, b2c, wh0, bh0c, wh1, bh1c, bpc):
    nsteps = 5
    const = lambda i: (0, 0)
    return pl.pallas_call(
        _encoder_body,
        grid=(nsteps,),
        in_specs=[
            pl.BlockSpec((40, N_NODES), lambda i: (i, 0)),
            pl.BlockSpec((N_NODES, 8), const),
            pl.BlockSpec((200, 1), const),
            pl.BlockSpec((NLAT, 200), const),
            pl.BlockSpec((NLAT, 1), const),
            pl.BlockSpec((128, NLAT), const),
            pl.BlockSpec((128, 1), const),
            pl.BlockSpec((64, 128), const),
            pl.BlockSpec((64, 1), const),
            pl.BlockSpec((1, 1), const),
        ],
        out_specs=[
            pl.BlockSpec((64, 128), const),
            pl.BlockSpec((128, 8), const),
        ],
        out_shape=[
            jax.ShapeDtypeStruct((64, 128), F32),
            jax.ShapeDtypeStruct((128, 8), F32),
        ],
        scratch_shapes=[pltpu.VMEM((200, 8), F32)],
    )(w1, xt, b1c, w2, b2c, wh0, bh0c, wh1, bh1c, bpc)


def _sc_body(wdec_hbm, nid2_hbm, dist_hbm, p_hbm, q_hbm,
             idx_v, dist_v, rows_v, p_v, q_v, sem):
    c = lax.axis_index("c")
    s = lax.axis_index("s")
    wid = s * 2 + c
    nt = jnp.where(wid < SC_NCHUNKS % 32,
                   SC_NCHUNKS // 32 + 1, SC_NCHUNKS // 32)

    @pl.loop(0, nt, unroll=1)
    def _chunk(t):
        ci = wid + 32 * t
        base = ci * SC_CHUNK
        row0 = ci * (SC_CHUNK * K // 128)
        pltpu.sync_copy(nid2_hbm.at[pl.ds(row0, SC_CHUNK * K // 128)], idx_v)
        pltpu.sync_copy(dist_hbm.at[pl.ds(base * K, SC_CHUNK * K)], dist_v)
        descs = [
            pltpu.async_copy(wdec_hbm.at[idx_v.at[g]],
                             rows_v.at[pl.ds(g * 128, 128)], sem)
            for g in range(SC_CHUNK * K // 128)
        ]
        for dsc in descs:
            dsc.wait()

        @pl.loop(0, SC_CHUNK, unroll=1)
        def _node(i):
            dvec = dist_v[pl.ds(i * K, K)]
            d2v = dvec * dvec
            p = jnp.zeros((16,), F32)
            q = jnp.zeros((16,), F32)
            for k in range(K):
                row = rows_v[i * K + k, :]
                p = p + row
                q = q + d2v[k] * row
            p_v[i, :] = p
            q_v[i, :] = q

        pltpu.sync_copy(p_v, p_hbm.at[pl.ds(base, SC_CHUNK)])
        pltpu.sync_copy(q_v, q_hbm.at[pl.ds(base, SC_CHUNK)])


def _sc_call(wdec_pad, nid2, distflat):
    mesh = plsc.VectorSubcoreMesh(core_axis_name="c", subcore_axis_name="s")
    assert SC_CHUNK * K // 128 % 8 == 0
    f = functools.partial(
        pl.kernel,
        out_type=[
            jax.ShapeDtypeStruct((NPAD, K), F32),
            jax.ShapeDtypeStruct((NPAD, K), F32),
        ],
        mesh=mesh,
        compiler_params=pltpu.CompilerParams(use_tc_tiling_on_sc=False),
        scratch_types=[
            pltpu.VMEM((SC_CHUNK * K // 128, 128), jnp.int32),
            pltpu.VMEM((SC_CHUNK * K,), F32),
            pltpu.VMEM((SC_CHUNK * K, K), F32),
            pltpu.VMEM((SC_CHUNK, K), F32),
            pltpu.VMEM((SC_CHUNK, K), F32),
            pltpu.SemaphoreType.DMA,
        ],
    )(_sc_body)
    return f(wdec_pad, nid2, distflat)


def _main_body(p_ref, q_ref, dist_ref, nid_ref, lab_ref, bdec_ref,
               vt_ref, e_ref, wdec_ref, out_ref):
    d = dist_ref[...]
    d2 = d * d
    ones16 = jnp.ones((K, 1), F32)
    s_sum = jnp.dot(d2, ones16, preferred_element_type=F32)
    lane64 = lax.broadcasted_iota(jnp.int32, (NB, 64), 1)
    oh = (lab_ref[...] == lane64).astype(F32)
    v = jnp.dot(oh, vt_ref[...], preferred_element_type=F32)
    sub16 = lax.broadcasted_iota(jnp.int32, (16, 128), 0)
    lane128 = lax.broadcasted_iota(jnp.int32, (16, 128), 1)
    repp = ((lane128 % 32) == sub16).astype(F32)
    clip = jnp.max(d2) * jnp.max(v) > 1.0

    @pl.when(jnp.logical_not(clip))
    def _fast():
        prep = jnp.dot(p_ref[...], repp, preferred_element_type=F32)
        qrep = jnp.dot(q_ref[...], repp, preferred_element_type=F32)
        conv = (prep - v * qrep) / (float(K) - v * s_sum)
        out_ref[...] = jnp.dot(conv, e_ref[...],
                               preferred_element_type=F32) + bdec_ref[...]

    @pl.when(clip)
    def _slow():
        accA = jnp.zeros((NB, 128), F32)
        accZ = jnp.zeros((NB, 128), F32)
        nid = nid_ref[...]
        lanech = lax.broadcasted_iota(jnp.int32, (NB, NCHUNK), 1)
        for k in range(K):
            idxk = nid[:, k:k + 1]

            def gk_body(cc, acc):
                wrow = wdec_ref[pl.ds(cc * NCHUNK, NCHUNK), :]
                ohc = (idxk == (cc * NCHUNK + lanech)).astype(F32)
                return acc + jnp.dot(ohc, wrow, preferred_element_type=F32)

            gk = lax.fori_loop(0, NPAD // NCHUNK, gk_body,
                               jnp.zeros((NB, K), F32))
            grep = jnp.dot(gk, repp, preferred_element_type=F32)
            win = jnp.maximum(1.0 - d2[:, k:k + 1] * v, 0.0)
            accZ = accZ + win
            accA = accA + win * grep
        conv = accA / accZ
        out_ref[...] = jnp.dot(conv, e_ref[...],
                               preferred_element_type=F32) + bdec_ref[...]


def _main_call(p_arr, q_arr, dist_pad, nid_pad, lab2, bdec2, vt, e_mat, wdec_pad):
    const = lambda i: (0, 0)
    blk = lambda i: (i, 0)
    return pl.pallas_call(
        _main_body,
        grid=(NPAD // NB,),
        in_specs=[
            pl.BlockSpec((NB, K), blk),
            pl.BlockSpec((NB, K), blk),
            pl.BlockSpec((NB, K), blk),
            pl.BlockSpec((NB, K), blk),
            pl.BlockSpec((NB, 1), blk),
            pl.BlockSpec((NB, 1), blk),
            pl.BlockSpec((64, 128), const),
            pl.BlockSpec((128, 8), const),
            pl.BlockSpec((NPAD, K), const),
        ],
        out_specs=pl.BlockSpec((NB, 8), blk),
        out_shape=jax.ShapeDtypeStruct((NPAD, 8), F32),
    )(p_arr, q_arr, dist_pad, nid_pad, lab2, bdec2, vt, e_mat, wdec_pad)


def kernel(x, W1, b1, W2, b2, Wdec, bdec, Wh0, bh0, Wh1, bh1, Bp,
           neighbour_distance, neighbour_id, clustering_labels):
    pad = NPAD - N_NODES
    xt = jnp.pad(x, ((0, 8 - NBATCH), (0, 0))).T
    wdec_pad = jnp.pad(Wdec, ((0, pad), (0, K - NLAT)))
    nid_pad = jnp.pad(neighbour_id, ((0, pad), (0, 0)))
    nid2 = nid_pad.reshape(NPAD * K // 128, 128)
    dist_pad = jnp.pad(neighbour_distance, ((0, pad), (0, 0)))
    distflat = dist_pad.reshape(NPAD * K)
    lab2 = jnp.pad(clustering_labels, (0, pad)).reshape(NPAD, 1)
    bdec2 = jnp.pad(bdec, (0, pad)).reshape(NPAD, 1)

    vt, e_mat = _encoder_call(
        W1, xt, b1.reshape(200, 1), W2, b2.reshape(NLAT, 1),
        Wh0, bh0.reshape(128, 1), Wh1, bh1.reshape(64, 1), Bp.reshape(1, 1))
    p_arr, q_arr = _sc_call(wdec_pad, nid2, distflat)
    out_t = _main_call(p_arr, q_arr, dist_pad, nid_pad, lab2, bdec2,
                       vt, e_mat, wdec_pad)
    return out_t[:N_NODES, :NBATCH].T
